# Initial kernel scaffold; baseline (speedup 1.0000x reference)
import functools

import jax
import jax.numpy as jnp
from jax import lax
from jax.experimental import pallas as pl
from jax.experimental.pallas import tpu as pltpu
from jax.experimental.pallas import tpu_sc as plsc

VOCAB = 1000000
DIM = 64
WORLD_SIZE = 4
RANK = 0
PART = VOCAB // WORLD_SIZE
START = RANK * PART
END = START + PART

NUM_CORES = 2
NUM_SUBCORES = 16
NUM_WORKERS = NUM_CORES * NUM_SUBCORES
LANES = 16

ZERO_ROW = PART
PAD_ROWS = 8

CHUNK = 1024


def _sc_gather(n_idx):
    per_w = n_idx // NUM_WORKERS
    n_chunks = per_w // CHUNK
    assert per_w % CHUNK == 0 and per_w % LANES == 0

    mesh = plsc.VectorSubcoreMesh(core_axis_name="c", subcore_axis_name="s")

    @functools.partial(
        pl.kernel,
        out_type=jax.ShapeDtypeStruct((n_idx, DIM), jnp.float32),
        mesh=mesh,
        scratch_types=[
            pltpu.VMEM((per_w,), jnp.int32),
            pltpu.VMEM((CHUNK, DIM), jnp.float32),
            pltpu.SemaphoreType.DMA,
        ],
    )
    def k(idx_hbm, table_hbm, out_hbm, idx_v, rows_v, sem):
        wid = lax.axis_index("s") * NUM_CORES + lax.axis_index("c")
        base = wid * per_w

        pltpu.sync_copy(idx_hbm.at[pl.ds(base, per_w)], idx_v)

        def remap(i, carry):
            v = idx_v[pl.ds(i * LANES, LANES)]
            m = (v < START) | (v >= END)
            idx_v[pl.ds(i * LANES, LANES)] = jnp.where(m, ZERO_ROW, v - START)
            return carry

        lax.fori_loop(0, per_w // LANES, remap, 0, unroll=4)

        def body(c, carry):
            pltpu.async_copy(
                table_hbm.at[idx_v.at[pl.ds(c * CHUNK, CHUNK)]], rows_v, sem
            ).wait()
            pltpu.sync_copy(rows_v, out_hbm.at[pl.ds(base + c * CHUNK, CHUNK)])
            return carry

        lax.fori_loop(0, n_chunks, body, 0)

    return k


def kernel(x, weight):
    n_idx = x.shape[0] * x.shape[1]
    x_flat = x.reshape(n_idx).astype(jnp.int32)
    table = jnp.concatenate(
        [weight, jnp.zeros((PAD_ROWS, DIM), jnp.float32)], axis=0
    )
    out = _sc_gather(n_idx)(x_flat, table)
    return out.reshape(x.shape[0], x.shape[1], DIM)


# speedup vs baseline: 2.0332x; 2.0332x over previous
import functools

import jax
import jax.numpy as jnp
from jax import lax
from jax.experimental import pallas as pl
from jax.experimental.pallas import tpu as pltpu
from jax.experimental.pallas import tpu_sc as plsc

VOCAB = 1000000
DIM = 64
WORLD_SIZE = 4
RANK = 0
PART = VOCAB // WORLD_SIZE
START = RANK * PART
END = START + PART

NUM_CORES = 2
NUM_SUBCORES = 16
NUM_WORKERS = NUM_CORES * NUM_SUBCORES
LANES = 16

ZERO_ROW = PART
PAD_ROWS = 8

CHUNK = 1024


def _sc_gather(n_idx):
    per_w = n_idx // NUM_WORKERS
    n_chunks = per_w // CHUNK
    assert per_w % CHUNK == 0 and per_w % LANES == 0

    mesh = plsc.VectorSubcoreMesh(core_axis_name="c", subcore_axis_name="s")

    @functools.partial(
        pl.kernel,
        out_type=jax.ShapeDtypeStruct((n_idx, DIM), jnp.float32),
        mesh=mesh,
        scratch_types=[
            pltpu.VMEM((per_w,), jnp.int32),
            pltpu.VMEM((CHUNK, DIM), jnp.float32),
            pltpu.SemaphoreType.DMA,
        ],
        compiler_params=pltpu.CompilerParams(use_tc_tiling_on_sc=False),
    )
    def k(idx_hbm, table_hbm, out_hbm, idx_v, rows_v, sem):
        wid = lax.axis_index("s") * NUM_CORES + lax.axis_index("c")
        base = wid * per_w

        pltpu.sync_copy(idx_hbm.at[pl.ds(base, per_w)], idx_v)

        def remap(i, carry):
            v = idx_v[pl.ds(i * LANES, LANES)]
            m = (v < START) | (v >= END)
            idx_v[pl.ds(i * LANES, LANES)] = jnp.where(m, ZERO_ROW, v - START)
            return carry

        lax.fori_loop(0, per_w // LANES, remap, 0, unroll=4)

        def body(c, carry):
            pltpu.async_copy(
                table_hbm.at[idx_v.at[pl.ds(c * CHUNK, CHUNK)]], rows_v, sem
            ).wait()
            pltpu.sync_copy(rows_v, out_hbm.at[pl.ds(base + c * CHUNK, CHUNK)])
            return carry

        lax.fori_loop(0, n_chunks, body, 0)

    return k


def kernel(x, weight):
    n_idx = x.shape[0] * x.shape[1]
    x_flat = x.reshape(n_idx).astype(jnp.int32)
    table = jnp.concatenate(
        [weight, jnp.zeros((PAD_ROWS, DIM), jnp.float32)], axis=0
    )
    out = _sc_gather(n_idx)(x_flat, table)
    return out.reshape(x.shape[0], x.shape[1], DIM)
